# baseline (speedup 1.0000x reference)
import functools

import jax
import jax.numpy as jnp
from jax import lax
from jax.experimental import pallas as pl
from jax.experimental.pallas import tpu as pltpu
from jax.experimental.pallas import tpu_sc as plsc

B = 1024
OBS = 1024
HID = 1024
E = 8
ACT = 16

T = 128
NT = B // T + E - 1
NTP = 16
NP = NT * T
PW = 128

_SC = plsc.get_sparse_core_info()
_NC, _NS = _SC.num_cores, _SC.num_subcores
NW = _NC * _NS
RPW = B // NW
LANES = 16

_F = jnp.float32


GH = 8
HB = B // GH


def _heads_body(obs_ref, dones_ref, eo_ref, Wcat_ref, bcat_ref,
                ma_ref, mv_ref, mlp_ref, tp_ref, pos_ref, texp_ref, allh_s):
    t = pl.program_id(0)
    allh_s[pl.ds(t * HB, HB), :] = (
        jnp.dot(obs_ref[...], Wcat_ref[...], preferred_element_type=_F)
        + bcat_ref[...])

    @pl.when(t == GH - 1)
    def _finish():
        _heads_finish(dones_ref, eo_ref, ma_ref, mv_ref, mlp_ref, tp_ref,
                      pos_ref, texp_ref, allh_s)


def _heads_finish(dones_ref, eo_ref, ma_ref, mv_ref, mlp_ref, tp_ref,
                  pos_ref, texp_ref, allh_s):
    eyeB = (lax.broadcasted_iota(jnp.int32, (B, B), 0)
            == lax.broadcasted_iota(jnp.int32, (B, B), 1)).astype(_F)
    eo_row = eo_ref[...].reshape(1, B).astype(_F)
    dones_row = dones_ref[...].reshape(1, B).astype(_F)
    eo_col = lax.dot_general(eyeB, eo_row, (((1,), (1,)), ((), ())),
                             preferred_element_type=_F).astype(jnp.int32)
    dones_col = lax.dot_general(eyeB, dones_row, (((1,), (1,)), ((), ())),
                                preferred_element_type=_F)
    idxBE = lax.broadcasted_iota(jnp.int32, (B, E), 1)
    allh = allh_s[...]
    meta_logits = allh[:, 0:E]
    term_logits = allh[:, E:2 * E]
    meta_values = allh[:, 2 * E:2 * E + 1]
    m = jnp.max(meta_logits, axis=-1, keepdims=True)
    meta_actions = jnp.min(jnp.where(meta_logits == m, idxBE, E),
                           axis=-1, keepdims=True)
    meta_log_probs = -jnp.log(jnp.sum(jnp.exp(meta_logits - m),
                                      axis=-1, keepdims=True))
    tp_all = jax.nn.sigmoid(term_logits)
    termination_probs = jnp.sum(jnp.where(idxBE == eo_col, tp_all, 0.0),
                                axis=-1, keepdims=True)
    terminates = (dones_col != 0) | (termination_probs > 0.5)
    new_option = jnp.where(terminates, meta_actions, eo_col)
    ma_ref[...] = meta_actions
    mv_ref[...] = meta_values
    mlp_ref[...] = meta_log_probs
    tp_ref[...] = termination_probs

    no_row = lax.dot_general(new_option.astype(_F), eyeB,
                             (((0,), (0,)), ((), ())),
                             preferred_element_type=_F)
    idxE = lax.broadcasted_iota(jnp.int32, (E, B), 0)
    maskT = (no_row == idxE.astype(_F)).astype(_F)
    counts = jnp.sum(maskT, axis=1, keepdims=True)
    upper = (lax.broadcasted_iota(jnp.int32, (B, B), 0)
             < lax.broadcasted_iota(jnp.int32, (B, B), 1)).astype(_F)
    rank = jnp.dot(maskT, upper, preferred_element_type=_F)
    tiles = jnp.floor((counts + (T - 1)) * (1.0 / T))
    before = (lax.broadcasted_iota(jnp.int32, (E, E), 1)
              < lax.broadcasted_iota(jnp.int32, (E, E), 0)).astype(_F)
    tbase = jnp.dot(before, tiles, preferred_element_type=_F)
    pos_f = jnp.sum(maskT * (T * tbase + rank), axis=0, keepdims=True)
    pos_ref[...] = pos_f.astype(jnp.int32)

    iotaEc = lax.broadcasted_iota(jnp.int32, (E, 1), 0).astype(_F)
    total = jnp.sum(tiles, axis=0, keepdims=True)
    e_last = jnp.max(jnp.where(tiles > 0, iotaEc, -1.0),
                     axis=0, keepdims=True)
    tt = lax.broadcasted_iota(jnp.int32, (E, NTP), 1).astype(_F)
    live = (tt >= tbase) & (tt < tbase + tiles)
    texp_live = jnp.sum(jnp.where(live, iotaEc, 0.0), axis=0, keepdims=True)
    dead = (lax.broadcasted_iota(jnp.int32, (1, NTP), 1).astype(_F) >= total)
    texp_row = jnp.where(dead, e_last, texp_live)
    is_last = (lax.broadcasted_iota(jnp.int32, (1, NTP), 1) == NTP - 1)
    texp_ref[...] = jnp.where(is_last, total, texp_row).astype(jnp.int32)


def _heads_call(observation, dones_i, eo, Wcat, bcat):
    b1spec = lambda r: pl.BlockSpec(r, lambda t: (0,) * len(r))
    return pl.pallas_call(
        _heads_body,
        grid=(GH,),
        in_specs=[
            pl.BlockSpec((HB, OBS), lambda t: (t, 0)),
            b1spec((B,)), b1spec((B,)),
            b1spec((OBS, 2 * E + 1)), b1spec((1, 2 * E + 1)),
        ],
        out_specs=[
            b1spec((B, 1)), b1spec((B, 1)), b1spec((B, 1)), b1spec((B, 1)),
            b1spec((1, B)), b1spec((1, NTP)),
        ],
        scratch_shapes=[pltpu.VMEM((B, 2 * E + 1), _F)],
        out_shape=[
            jax.ShapeDtypeStruct((B, 1), jnp.int32),
            jax.ShapeDtypeStruct((B, 1), _F),
            jax.ShapeDtypeStruct((B, 1), _F),
            jax.ShapeDtypeStruct((B, 1), _F),
            jax.ShapeDtypeStruct((1, B), jnp.int32),
            jax.ShapeDtypeStruct((1, NTP), jnp.int32),
        ],
    )(observation, dones_i, eo, Wcat, bcat)


def _expert_body(te_ref, sobs_ref, W1_ref, b1_ref, W2_ref, b2_ref,
                 Wv_ref, bv_ref, out_ref):
    t = pl.program_id(0)

    @pl.when(t < te_ref[NTP - 1])
    def _live():
        e = te_ref[t]
        sel8 = lax.broadcasted_iota(jnp.int32, (E, 1), 0) == e
        b1row = jnp.sum(jnp.where(sel8, b1_ref[...], 0.0), axis=0,
                        keepdims=True)
        b2row = jnp.sum(jnp.where(sel8, b2_ref[...], 0.0), axis=0,
                        keepdims=True)
        wvrow = jnp.sum(jnp.where(sel8, Wv_ref[...], 0.0), axis=0,
                        keepdims=True)
        x = sobs_ref[...]
        h = jnp.maximum(
            jnp.dot(x, W1_ref[0], preferred_element_type=_F) + b1row, 0.0)
        logits = jnp.dot(h, W2_ref[0], preferred_element_type=_F) + b2row
        vals = jnp.sum(h * wvrow, axis=1, keepdims=True) + bv_ref[e]
        m = jnp.max(logits, axis=-1, keepdims=True)
        idx = lax.broadcasted_iota(jnp.int32, (T, ACT), 1)
        acts = jnp.min(jnp.where(logits == m, idx, ACT), axis=-1,
                       keepdims=True)
        lps = -jnp.log(jnp.sum(jnp.exp(logits - m), axis=-1, keepdims=True))
        out_ref[...] = jnp.concatenate(
            [acts.astype(_F), vals, lps, jnp.zeros((T, PW - 3), _F)], axis=1)


def _experts_call(te, sorted_obs, W1, b1, W2, b2, Wv, bv):
    grid_spec = pltpu.PrefetchScalarGridSpec(
        num_scalar_prefetch=1,
        grid=(NT,),
        in_specs=[
            pl.BlockSpec((T, OBS),
                         lambda t, te: (jnp.where(t < te[NTP - 1], t, 0), 0)),
            pl.BlockSpec((1, OBS, HID), lambda t, te: (te[t], 0, 0)),
            pl.BlockSpec((E, HID), lambda t, te: (0, 0)),
            pl.BlockSpec((1, HID, ACT), lambda t, te: (te[t], 0, 0)),
            pl.BlockSpec((E, ACT), lambda t, te: (0, 0)),
            pl.BlockSpec((E, HID), lambda t, te: (0, 0)),
            pl.BlockSpec(memory_space=pltpu.SMEM),
        ],
        out_specs=pl.BlockSpec((T, PW), lambda t, te: (t, 0)),
    )
    return pl.pallas_call(
        _expert_body,
        grid_spec=grid_spec,
        out_shape=jax.ShapeDtypeStruct((NP, PW), _F),
    )(te, sorted_obs, W1, b1, W2, b2, Wv, bv)


_sc_mesh = plsc.VectorSubcoreMesh(core_axis_name="c", subcore_axis_name="s")


NCH = 4
CH = RPW // NCH


@functools.partial(
    pl.kernel, mesh=_sc_mesh,
    out_type=jax.ShapeDtypeStruct((NP, OBS), _F),
    scratch_types=[
        pltpu.VMEM((NCH, CH), jnp.int32),
        pltpu.VMEM((RPW, OBS), _F),
        pltpu.SemaphoreType.DMA,
        pltpu.SemaphoreType.DMA,
        pltpu.SemaphoreType.DMA,
        pltpu.SemaphoreType.DMA,
        pltpu.SemaphoreType.DMA,
    ],
)
def _sc_scatter_obs(pos_hbm, obs_hbm, out_hbm, idx_v, rows_v, lsem0, lsem1,
                    lsem2, lsem3, ssem):
    wid = lax.axis_index("s") * _NC + lax.axis_index("c")
    base = wid * RPW
    loads = []
    for j, lsem in enumerate((lsem0, lsem1, lsem2, lsem3)):
        pltpu.sync_copy(pos_hbm.at[0, pl.ds(base + j * CH, CH)], idx_v.at[j])
        loads.append(pltpu.async_copy(
            obs_hbm.at[pl.ds(base + j * CH, CH)],
            rows_v.at[pl.ds(j * CH, CH)], lsem))
    stores = []
    for j in range(NCH):
        loads[j].wait()
        stores.append(pltpu.async_copy(
            rows_v.at[pl.ds(j * CH, CH)], out_hbm.at[idx_v.at[j]], ssem))
    for s in stores:
        s.wait()


@functools.partial(
    pl.kernel, mesh=_sc_mesh,
    out_type=[
        jax.ShapeDtypeStruct((B,), jnp.int32),
        jax.ShapeDtypeStruct((B,), _F),
        jax.ShapeDtypeStruct((B,), _F),
    ],
    scratch_types=[
        pltpu.VMEM((RPW,), jnp.int32),
        pltpu.VMEM((RPW,), jnp.int32),
        pltpu.VMEM((RPW,), jnp.int32),
        pltpu.VMEM((RPW,), jnp.int32),
        pltpu.VMEM((RPW,), _F),
        pltpu.VMEM((RPW,), _F),
        pltpu.VMEM((RPW,), _F),
        pltpu.VMEM((RPW,), jnp.int32),
        pltpu.SemaphoreType.DMA,
    ],
)
def _sc_gather_out(pos_hbm, packed_hbm, act_hbm, val_hbm, lp_hbm,
                   idx_v, ai_v, vi_v, li_v, af_v, vf_v, lf_v, a_v, sem):
    wid = lax.axis_index("s") * _NC + lax.axis_index("c")
    base = wid * RPW
    pltpu.sync_copy(pos_hbm.at[0, pl.ds(base, RPW)], idx_v)
    for j in range(RPW // LANES):
        sl = pl.ds(j * LANES, LANES)
        flat = idx_v[sl] * PW
        ai_v[sl] = flat
        vi_v[sl] = flat + 1
        li_v[sl] = flat + 2
    pltpu.async_copy(packed_hbm.at[ai_v], af_v, sem).wait()
    pltpu.async_copy(packed_hbm.at[vi_v], vf_v, sem).wait()
    pltpu.async_copy(packed_hbm.at[li_v], lf_v, sem).wait()
    for j in range(RPW // LANES):
        sl = pl.ds(j * LANES, LANES)
        a_v[sl] = af_v[sl].astype(jnp.int32)
    pltpu.sync_copy(a_v, act_hbm.at[pl.ds(base, RPW)])
    pltpu.sync_copy(vf_v, val_hbm.at[pl.ds(base, RPW)])
    pltpu.sync_copy(lf_v, lp_hbm.at[pl.ds(base, RPW)])


def kernel(observation, dones, executing_option, W_meta, b_meta, W_mv, b_mv,
           W_term, b_term, W1, b1, W2, b2, Wv, bv):
    dones_i = dones
    eo = executing_option.astype(jnp.int32)
    Wcat = jnp.concatenate([W_meta, W_term, W_mv], axis=1)
    bcat = jnp.concatenate([b_meta, b_term, b_mv]).reshape(1, 2 * E + 1)

    ma, mv, mlp, tp, pos, texp = _heads_call(
        observation, dones_i, eo, Wcat, bcat)
    te = texp.reshape(NTP)

    sorted_obs = _sc_scatter_obs(pos, observation)
    packed = _experts_call(te, sorted_obs, W1, b1, W2, b2, Wv, bv)
    actions, values, log_probs = _sc_gather_out(pos, packed.reshape(NP * PW))

    return (actions, values, log_probs, ma.reshape(B), mv.reshape(B),
            mlp.reshape(B), tp.reshape(B))


# speedup vs baseline: 1.0480x; 1.0480x over previous
import functools

import jax
import jax.numpy as jnp
from jax import lax
from jax.experimental import pallas as pl
from jax.experimental.pallas import tpu as pltpu
from jax.experimental.pallas import tpu_sc as plsc

B = 1024
OBS = 1024
HID = 1024
E = 8
ACT = 16

T = 128
NT = B // T + E - 1
NTP = 16
NP = NT * T
PW = 128

_SC = plsc.get_sparse_core_info()
_NC, _NS = _SC.num_cores, _SC.num_subcores
NW = _NC * _NS
RPW = B // NW
LANES = 16

_F = jnp.float32


def _heads_body(obs_ref, dones_ref, eo_ref, Wcat_ref, bcat_ref,
                ma_ref, mv_ref, mlp_ref, tp_ref, pos_ref, texp_ref):
    eyeB = (lax.broadcasted_iota(jnp.int32, (B, B), 0)
            == lax.broadcasted_iota(jnp.int32, (B, B), 1)).astype(_F)
    eo_row = eo_ref[...].reshape(1, B).astype(_F)
    dones_row = dones_ref[...].reshape(1, B).astype(_F)
    eo_col = lax.dot_general(eyeB, eo_row, (((1,), (1,)), ((), ())),
                             preferred_element_type=_F).astype(jnp.int32)
    dones_col = lax.dot_general(eyeB, dones_row, (((1,), (1,)), ((), ())),
                                preferred_element_type=_F)
    idxBE = lax.broadcasted_iota(jnp.int32, (B, E), 1)
    allh = jnp.dot(obs_ref[...], Wcat_ref[...],
                   preferred_element_type=_F) + bcat_ref[...]
    meta_logits = allh[:, 0:E]
    term_logits = allh[:, E:2 * E]
    meta_values = allh[:, 2 * E:2 * E + 1]
    m = jnp.max(meta_logits, axis=-1, keepdims=True)
    meta_actions = jnp.min(jnp.where(meta_logits == m, idxBE, E),
                           axis=-1, keepdims=True)
    meta_log_probs = -jnp.log(jnp.sum(jnp.exp(meta_logits - m),
                                      axis=-1, keepdims=True))
    tp_all = jax.nn.sigmoid(term_logits)
    termination_probs = jnp.sum(jnp.where(idxBE == eo_col, tp_all, 0.0),
                                axis=-1, keepdims=True)
    terminates = (dones_col != 0) | (termination_probs > 0.5)
    new_option = jnp.where(terminates, meta_actions, eo_col)
    ma_ref[...] = meta_actions
    mv_ref[...] = meta_values
    mlp_ref[...] = meta_log_probs
    tp_ref[...] = termination_probs

    no_row = lax.dot_general(new_option.astype(_F), eyeB,
                             (((0,), (0,)), ((), ())),
                             preferred_element_type=_F)
    idxE = lax.broadcasted_iota(jnp.int32, (E, B), 0)
    maskT = (no_row == idxE.astype(_F)).astype(_F)
    counts = jnp.sum(maskT, axis=1, keepdims=True)
    upper = (lax.broadcasted_iota(jnp.int32, (B, B), 0)
             < lax.broadcasted_iota(jnp.int32, (B, B), 1)).astype(_F)
    rank = jnp.dot(maskT, upper, preferred_element_type=_F)
    tiles = jnp.floor((counts + (T - 1)) * (1.0 / T))
    before = (lax.broadcasted_iota(jnp.int32, (E, E), 1)
              < lax.broadcasted_iota(jnp.int32, (E, E), 0)).astype(_F)
    tbase = jnp.dot(before, tiles, preferred_element_type=_F)
    pos_f = jnp.sum(maskT * (T * tbase + rank), axis=0, keepdims=True)
    pos_ref[...] = pos_f.astype(jnp.int32)

    iotaEc = lax.broadcasted_iota(jnp.int32, (E, 1), 0).astype(_F)
    total = jnp.sum(tiles, axis=0, keepdims=True)
    e_last = jnp.max(jnp.where(tiles > 0, iotaEc, -1.0),
                     axis=0, keepdims=True)
    tt = lax.broadcasted_iota(jnp.int32, (E, NTP), 1).astype(_F)
    live = (tt >= tbase) & (tt < tbase + tiles)
    texp_live = jnp.sum(jnp.where(live, iotaEc, 0.0), axis=0, keepdims=True)
    dead = (lax.broadcasted_iota(jnp.int32, (1, NTP), 1).astype(_F) >= total)
    texp_row = jnp.where(dead, e_last, texp_live)
    is_last = (lax.broadcasted_iota(jnp.int32, (1, NTP), 1) == NTP - 1)
    texp_ref[...] = jnp.where(is_last, total, texp_row).astype(jnp.int32)


def _heads_call(observation, dones_i, eo, Wcat, bcat):
    b1spec = lambda r: pl.BlockSpec(r, lambda: (0,) * len(r))
    return pl.pallas_call(
        _heads_body,
        in_specs=[
            b1spec((B, OBS)), b1spec((B,)), b1spec((B,)),
            b1spec((OBS, 2 * E + 1)), b1spec((1, 2 * E + 1)),
        ],
        out_specs=[
            b1spec((B, 1)), b1spec((B, 1)), b1spec((B, 1)), b1spec((B, 1)),
            b1spec((1, B)), b1spec((1, NTP)),
        ],
        out_shape=[
            jax.ShapeDtypeStruct((B, 1), jnp.int32),
            jax.ShapeDtypeStruct((B, 1), _F),
            jax.ShapeDtypeStruct((B, 1), _F),
            jax.ShapeDtypeStruct((B, 1), _F),
            jax.ShapeDtypeStruct((1, B), jnp.int32),
            jax.ShapeDtypeStruct((1, NTP), jnp.int32),
        ],
    )(observation, dones_i, eo, Wcat, bcat)


def _expert_body(te_ref, sobs_ref, W1_ref, b1_ref, W2_ref, b2_ref,
                 Wv_ref, bv_ref, out_ref):
    t = pl.program_id(0)

    @pl.when(t < te_ref[NTP - 1])
    def _live():
        e = te_ref[t]
        sel8 = lax.broadcasted_iota(jnp.int32, (E, 1), 0) == e
        b1row = jnp.sum(jnp.where(sel8, b1_ref[...], 0.0), axis=0,
                        keepdims=True)
        b2row = jnp.sum(jnp.where(sel8, b2_ref[...], 0.0), axis=0,
                        keepdims=True)
        wvrow = jnp.sum(jnp.where(sel8, Wv_ref[...], 0.0), axis=0,
                        keepdims=True)
        x = sobs_ref[...]
        h = jnp.maximum(
            jnp.dot(x, W1_ref[0], preferred_element_type=_F) + b1row, 0.0)
        logits = jnp.dot(h, W2_ref[0], preferred_element_type=_F) + b2row
        vals = jnp.sum(h * wvrow, axis=1, keepdims=True) + bv_ref[e]
        m = jnp.max(logits, axis=-1, keepdims=True)
        idx = lax.broadcasted_iota(jnp.int32, (T, ACT), 1)
        acts = jnp.min(jnp.where(logits == m, idx, ACT), axis=-1,
                       keepdims=True)
        lps = -jnp.log(jnp.sum(jnp.exp(logits - m), axis=-1, keepdims=True))
        out_ref[...] = jnp.concatenate(
            [acts.astype(_F), vals, lps, jnp.zeros((T, PW - 3), _F)], axis=1)


def _experts_call(te, sorted_obs, W1, b1, W2, b2, Wv, bv):
    grid_spec = pltpu.PrefetchScalarGridSpec(
        num_scalar_prefetch=1,
        grid=(NT,),
        in_specs=[
            pl.BlockSpec((T, OBS),
                         lambda t, te: (jnp.where(t < te[NTP - 1], t, 0), 0)),
            pl.BlockSpec((1, OBS, HID), lambda t, te: (te[t], 0, 0)),
            pl.BlockSpec((E, HID), lambda t, te: (0, 0)),
            pl.BlockSpec((1, HID, ACT), lambda t, te: (te[t], 0, 0)),
            pl.BlockSpec((E, ACT), lambda t, te: (0, 0)),
            pl.BlockSpec((E, HID), lambda t, te: (0, 0)),
            pl.BlockSpec(memory_space=pltpu.SMEM),
        ],
        out_specs=pl.BlockSpec((T, PW), lambda t, te: (t, 0)),
    )
    return pl.pallas_call(
        _expert_body,
        grid_spec=grid_spec,
        out_shape=jax.ShapeDtypeStruct((NP, PW), _F),
    )(te, sorted_obs, W1, b1, W2, b2, Wv, bv)


_sc_mesh = plsc.VectorSubcoreMesh(core_axis_name="c", subcore_axis_name="s")


NCH = 4
CH = RPW // NCH


@functools.partial(
    pl.kernel, mesh=_sc_mesh,
    out_type=jax.ShapeDtypeStruct((NP, OBS), _F),
    scratch_types=[
        pltpu.VMEM((NCH, CH), jnp.int32),
        pltpu.VMEM((RPW, OBS), _F),
        pltpu.SemaphoreType.DMA,
        pltpu.SemaphoreType.DMA,
        pltpu.SemaphoreType.DMA,
        pltpu.SemaphoreType.DMA,
        pltpu.SemaphoreType.DMA,
    ],
)
def _sc_scatter_obs(pos_hbm, obs_hbm, out_hbm, idx_v, rows_v, lsem0, lsem1,
                    lsem2, lsem3, ssem):
    wid = lax.axis_index("s") * _NC + lax.axis_index("c")
    base = wid * RPW
    loads = []
    for j, lsem in enumerate((lsem0, lsem1, lsem2, lsem3)):
        pltpu.sync_copy(pos_hbm.at[0, pl.ds(base + j * CH, CH)], idx_v.at[j])
        loads.append(pltpu.async_copy(
            obs_hbm.at[pl.ds(base + j * CH, CH)],
            rows_v.at[pl.ds(j * CH, CH)], lsem))
    stores = []
    for j in range(NCH):
        loads[j].wait()
        stores.append(pltpu.async_copy(
            rows_v.at[pl.ds(j * CH, CH)], out_hbm.at[idx_v.at[j]], ssem))
    for s in stores:
        s.wait()


@functools.partial(
    pl.kernel, mesh=_sc_mesh,
    out_type=[
        jax.ShapeDtypeStruct((B,), jnp.int32),
        jax.ShapeDtypeStruct((B,), _F),
        jax.ShapeDtypeStruct((B,), _F),
    ],
    scratch_types=[
        pltpu.VMEM((RPW,), jnp.int32),
        pltpu.VMEM((RPW,), jnp.int32),
        pltpu.VMEM((RPW,), jnp.int32),
        pltpu.VMEM((RPW,), jnp.int32),
        pltpu.VMEM((RPW,), _F),
        pltpu.VMEM((RPW,), _F),
        pltpu.VMEM((RPW,), _F),
        pltpu.VMEM((RPW,), jnp.int32),
        pltpu.SemaphoreType.DMA,
    ],
)
def _sc_gather_out(pos_hbm, packed_hbm, act_hbm, val_hbm, lp_hbm,
                   idx_v, ai_v, vi_v, li_v, af_v, vf_v, lf_v, a_v, sem):
    wid = lax.axis_index("s") * _NC + lax.axis_index("c")
    base = wid * RPW
    pltpu.sync_copy(pos_hbm.at[0, pl.ds(base, RPW)], idx_v)
    for j in range(RPW // LANES):
        sl = pl.ds(j * LANES, LANES)
        flat = idx_v[sl] * PW
        ai_v[sl] = flat
        vi_v[sl] = flat + 1
        li_v[sl] = flat + 2
    pltpu.async_copy(packed_hbm.at[ai_v], af_v, sem).wait()
    pltpu.async_copy(packed_hbm.at[vi_v], vf_v, sem).wait()
    pltpu.async_copy(packed_hbm.at[li_v], lf_v, sem).wait()
    for j in range(RPW // LANES):
        sl = pl.ds(j * LANES, LANES)
        a_v[sl] = af_v[sl].astype(jnp.int32)
    pltpu.sync_copy(a_v, act_hbm.at[pl.ds(base, RPW)])
    pltpu.sync_copy(vf_v, val_hbm.at[pl.ds(base, RPW)])
    pltpu.sync_copy(lf_v, lp_hbm.at[pl.ds(base, RPW)])


def kernel(observation, dones, executing_option, W_meta, b_meta, W_mv, b_mv,
           W_term, b_term, W1, b1, W2, b2, Wv, bv):
    dones_i = dones
    eo = executing_option.astype(jnp.int32)
    Wcat = jnp.concatenate([W_meta, W_term, W_mv], axis=1)
    bcat = jnp.concatenate([b_meta, b_term, b_mv]).reshape(1, 2 * E + 1)

    ma, mv, mlp, tp, pos, texp = _heads_call(
        observation, dones_i, eo, Wcat, bcat)
    te = texp.reshape(NTP)

    sorted_obs = _sc_scatter_obs(pos, observation)
    packed = _experts_call(te, sorted_obs, W1, b1, W2, b2, Wv, bv)
    actions, values, log_probs = _sc_gather_out(pos, packed.reshape(NP * PW))

    return (actions, values, log_probs, ma.reshape(B), mv.reshape(B),
            mlp.reshape(B), tp.reshape(B))


# speedup vs baseline: 1.0593x; 1.0108x over previous
import functools

import jax
import jax.numpy as jnp
from jax import lax
from jax.experimental import pallas as pl
from jax.experimental.pallas import tpu as pltpu
from jax.experimental.pallas import tpu_sc as plsc

B = 1024
OBS = 1024
HID = 1024
E = 8
ACT = 16

T = 128
NT = B // T + E - 1
NTP = 16
NP = NT * T
PW = 128

_SC = plsc.get_sparse_core_info()
_NC, _NS = _SC.num_cores, _SC.num_subcores
NW = _NC * _NS
RPW = B // NW
LANES = 16

_F = jnp.float32


def _heads_body(obs_ref, dones_ref, eo_ref, Wcat_ref, bcat_ref,
                ma_ref, mv_ref, mlp_ref, tp_ref, pos_ref, texp_ref):
    eyeB = (lax.broadcasted_iota(jnp.int32, (B, B), 0)
            == lax.broadcasted_iota(jnp.int32, (B, B), 1)).astype(_F)
    eo_row = eo_ref[...].reshape(1, B).astype(_F)
    dones_row = dones_ref[...].reshape(1, B).astype(_F)
    eo_col = lax.dot_general(eyeB, eo_row, (((1,), (1,)), ((), ())),
                             preferred_element_type=_F).astype(jnp.int32)
    dones_col = lax.dot_general(eyeB, dones_row, (((1,), (1,)), ((), ())),
                                preferred_element_type=_F)
    idxBE = lax.broadcasted_iota(jnp.int32, (B, E), 1)
    allh = jnp.dot(obs_ref[...], Wcat_ref[...],
                   preferred_element_type=_F) + bcat_ref[...]
    meta_logits = allh[:, 0:E]
    term_logits = allh[:, E:2 * E]
    meta_values = allh[:, 2 * E:2 * E + 1]
    m = jnp.max(meta_logits, axis=-1, keepdims=True)
    meta_actions = jnp.min(jnp.where(meta_logits == m, idxBE, E),
                           axis=-1, keepdims=True)
    meta_log_probs = -jnp.log(jnp.sum(jnp.exp(meta_logits - m),
                                      axis=-1, keepdims=True))
    tp_all = jax.nn.sigmoid(term_logits)
    termination_probs = jnp.sum(jnp.where(idxBE == eo_col, tp_all, 0.0),
                                axis=-1, keepdims=True)
    terminates = (dones_col != 0) | (termination_probs > 0.5)
    new_option = jnp.where(terminates, meta_actions, eo_col)
    ma_ref[...] = meta_actions
    mv_ref[...] = meta_values
    mlp_ref[...] = meta_log_probs
    tp_ref[...] = termination_probs

    no_row = lax.dot_general(new_option.astype(_F), eyeB,
                             (((0,), (0,)), ((), ())),
                             preferred_element_type=_F)
    idxE = lax.broadcasted_iota(jnp.int32, (E, B), 0)
    maskT = (no_row == idxE.astype(_F)).astype(_F)
    counts = jnp.sum(maskT, axis=1, keepdims=True)
    upper = (lax.broadcasted_iota(jnp.int32, (B, B), 0)
             < lax.broadcasted_iota(jnp.int32, (B, B), 1)).astype(_F)
    rank = jnp.dot(maskT, upper, preferred_element_type=_F)
    tiles = jnp.floor((counts + (T - 1)) * (1.0 / T))
    before = (lax.broadcasted_iota(jnp.int32, (E, E), 1)
              < lax.broadcasted_iota(jnp.int32, (E, E), 0)).astype(_F)
    tbase = jnp.dot(before, tiles, preferred_element_type=_F)
    pos_f = jnp.sum(maskT * (T * tbase + rank), axis=0, keepdims=True)
    pos_ref[...] = pos_f.astype(jnp.int32)

    iotaEc = lax.broadcasted_iota(jnp.int32, (E, 1), 0).astype(_F)
    total = jnp.sum(tiles, axis=0, keepdims=True)
    e_last = jnp.max(jnp.where(tiles > 0, iotaEc, -1.0),
                     axis=0, keepdims=True)
    tt = lax.broadcasted_iota(jnp.int32, (E, NTP), 1).astype(_F)
    live = (tt >= tbase) & (tt < tbase + tiles)
    texp_live = jnp.sum(jnp.where(live, iotaEc, 0.0), axis=0, keepdims=True)
    dead = (lax.broadcasted_iota(jnp.int32, (1, NTP), 1).astype(_F) >= total)
    texp_row = jnp.where(dead, e_last, texp_live)
    is_last = (lax.broadcasted_iota(jnp.int32, (1, NTP), 1) == NTP - 1)
    texp_ref[...] = jnp.where(is_last, total, texp_row).astype(jnp.int32)


def _heads_call(observation, dones_i, eo, Wcat, bcat):
    b1spec = lambda r: pl.BlockSpec(r, lambda: (0,) * len(r))
    return pl.pallas_call(
        _heads_body,
        in_specs=[
            b1spec((B, OBS)), b1spec((B,)), b1spec((B,)),
            b1spec((OBS, 2 * E + 1)), b1spec((1, 2 * E + 1)),
        ],
        out_specs=[
            b1spec((B, 1)), b1spec((B, 1)), b1spec((B, 1)), b1spec((B, 1)),
            b1spec((1, B)), b1spec((1, NTP)),
        ],
        out_shape=[
            jax.ShapeDtypeStruct((B, 1), jnp.int32),
            jax.ShapeDtypeStruct((B, 1), _F),
            jax.ShapeDtypeStruct((B, 1), _F),
            jax.ShapeDtypeStruct((B, 1), _F),
            jax.ShapeDtypeStruct((1, B), jnp.int32),
            jax.ShapeDtypeStruct((1, NTP), jnp.int32),
        ],
    )(observation, dones_i, eo, Wcat, bcat)


def _expert_body(te_ref, sobs_ref, W1_ref, b1_ref, W2_ref, b2_ref,
                 Wv_ref, bv_ref, out_ref):
    t = pl.program_id(0)

    @pl.when(t < te_ref[NTP - 1])
    def _live():
        e = te_ref[t]
        sel8 = lax.broadcasted_iota(jnp.int32, (E, 1), 0) == e
        b1row = jnp.sum(jnp.where(sel8, b1_ref[...], 0.0), axis=0,
                        keepdims=True)
        b2row = jnp.sum(jnp.where(sel8, b2_ref[...], 0.0), axis=0,
                        keepdims=True)
        wvrow = jnp.sum(jnp.where(sel8, Wv_ref[...], 0.0), axis=0,
                        keepdims=True)
        x = sobs_ref[...]
        h = jnp.maximum(
            jnp.dot(x, W1_ref[0], preferred_element_type=_F) + b1row, 0.0)
        logits = jnp.dot(h, W2_ref[0], preferred_element_type=_F) + b2row
        vals = jnp.sum(h * wvrow, axis=1, keepdims=True) + bv_ref[e]
        m = jnp.max(logits, axis=-1, keepdims=True)
        idx = lax.broadcasted_iota(jnp.int32, (T, ACT), 1)
        acts = jnp.min(jnp.where(logits == m, idx, ACT), axis=-1,
                       keepdims=True)
        lps = -jnp.log(jnp.sum(jnp.exp(logits - m), axis=-1, keepdims=True))
        out_ref[...] = jnp.concatenate(
            [acts.astype(_F), vals, lps, jnp.zeros((T, PW - 3), _F)], axis=1)


def _experts_call(te, sorted_obs, W1, b1, W2, b2, Wv, bv):
    grid_spec = pltpu.PrefetchScalarGridSpec(
        num_scalar_prefetch=1,
        grid=(NT,),
        in_specs=[
            pl.BlockSpec((T, OBS),
                         lambda t, te: (jnp.where(t < te[NTP - 1], t, 0), 0)),
            pl.BlockSpec((1, OBS, HID), lambda t, te: (te[t], 0, 0)),
            pl.BlockSpec((E, HID), lambda t, te: (0, 0)),
            pl.BlockSpec((1, HID, ACT), lambda t, te: (te[t], 0, 0)),
            pl.BlockSpec((E, ACT), lambda t, te: (0, 0)),
            pl.BlockSpec((E, HID), lambda t, te: (0, 0)),
            pl.BlockSpec(memory_space=pltpu.SMEM),
        ],
        out_specs=pl.BlockSpec((T, PW), lambda t, te: (t, 0)),
    )
    return pl.pallas_call(
        _expert_body,
        grid_spec=grid_spec,
        out_shape=jax.ShapeDtypeStruct((NP, PW), _F),
    )(te, sorted_obs, W1, b1, W2, b2, Wv, bv)


_sc_mesh = plsc.VectorSubcoreMesh(core_axis_name="c", subcore_axis_name="s")


NCH = 4
CH = RPW // NCH


@functools.partial(
    pl.kernel, mesh=_sc_mesh,
    out_type=jax.ShapeDtypeStruct((NP, OBS), _F),
    scratch_types=[
        pltpu.VMEM((NCH, CH), jnp.int32),
        pltpu.VMEM((RPW, OBS), _F),
        pltpu.SemaphoreType.DMA,
        pltpu.SemaphoreType.DMA,
        pltpu.SemaphoreType.DMA,
        pltpu.SemaphoreType.DMA,
        pltpu.SemaphoreType.DMA,
    ],
)
def _sc_scatter_obs(pos_hbm, obs_hbm, out_hbm, idx_v, rows_v, lsem0, lsem1,
                    lsem2, lsem3, ssem):
    wid = lax.axis_index("s") * _NC + lax.axis_index("c")
    base = wid * RPW
    loads = []
    for j, lsem in enumerate((lsem0, lsem1, lsem2, lsem3)):
        pltpu.sync_copy(pos_hbm.at[0, pl.ds(base + j * CH, CH)], idx_v.at[j])
        loads.append(pltpu.async_copy(
            obs_hbm.at[pl.ds(base + j * CH, CH)],
            rows_v.at[pl.ds(j * CH, CH)], lsem))
    stores = []
    for j in range(NCH):
        loads[j].wait()
        stores.append(pltpu.async_copy(
            rows_v.at[pl.ds(j * CH, CH)], out_hbm.at[idx_v.at[j]], ssem))
    for s in stores:
        s.wait()


@functools.partial(
    pl.kernel, mesh=_sc_mesh,
    out_type=[
        jax.ShapeDtypeStruct((B,), jnp.int32),
        jax.ShapeDtypeStruct((B,), _F),
        jax.ShapeDtypeStruct((B,), _F),
    ],
    scratch_types=[
        pltpu.VMEM((RPW,), jnp.int32),
        pltpu.VMEM((RPW,), jnp.int32),
        pltpu.VMEM((RPW,), jnp.int32),
        pltpu.VMEM((RPW,), jnp.int32),
        pltpu.VMEM((RPW,), _F),
        pltpu.VMEM((RPW,), _F),
        pltpu.VMEM((RPW,), _F),
        pltpu.VMEM((RPW,), jnp.int32),
        pltpu.SemaphoreType.DMA,
    ],
)
def _sc_gather_out(pos_hbm, packed_hbm, act_hbm, val_hbm, lp_hbm,
                   idx_v, ai_v, vi_v, li_v, af_v, vf_v, lf_v, a_v, sem):
    wid = lax.axis_index("s") * _NC + lax.axis_index("c")
    base = wid * RPW
    pltpu.sync_copy(pos_hbm.at[0, pl.ds(base, RPW)], idx_v)
    for j in range(RPW // LANES):
        sl = pl.ds(j * LANES, LANES)
        flat = idx_v[sl] * PW
        ai_v[sl] = flat
        vi_v[sl] = flat + 1
        li_v[sl] = flat + 2
    c1 = pltpu.async_copy(packed_hbm.at[ai_v], af_v, sem)
    c2 = pltpu.async_copy(packed_hbm.at[vi_v], vf_v, sem)
    c3 = pltpu.async_copy(packed_hbm.at[li_v], lf_v, sem)
    c1.wait()
    c2.wait()
    c3.wait()
    for j in range(RPW // LANES):
        sl = pl.ds(j * LANES, LANES)
        a_v[sl] = af_v[sl].astype(jnp.int32)
    pltpu.sync_copy(a_v, act_hbm.at[pl.ds(base, RPW)])
    pltpu.sync_copy(vf_v, val_hbm.at[pl.ds(base, RPW)])
    pltpu.sync_copy(lf_v, lp_hbm.at[pl.ds(base, RPW)])


def kernel(observation, dones, executing_option, W_meta, b_meta, W_mv, b_mv,
           W_term, b_term, W1, b1, W2, b2, Wv, bv):
    dones_i = dones
    eo = executing_option.astype(jnp.int32)
    Wcat = jnp.concatenate([W_meta, W_term, W_mv], axis=1)
    bcat = jnp.concatenate([b_meta, b_term, b_mv]).reshape(1, 2 * E + 1)

    ma, mv, mlp, tp, pos, texp = _heads_call(
        observation, dones_i, eo, Wcat, bcat)
    te = texp.reshape(NTP)

    sorted_obs = _sc_scatter_obs(pos, observation)
    packed = _experts_call(te, sorted_obs, W1, b1, W2, b2, Wv, bv)
    actions, values, log_probs = _sc_gather_out(pos, packed.reshape(NP * PW))

    return (actions, values, log_probs, ma.reshape(B), mv.reshape(B),
            mlp.reshape(B), tp.reshape(B))
